# baseline (speedup 1.0000x reference)
import jax
import jax.numpy as jnp
from jax import lax
from jax.experimental import pallas as pl
from jax.experimental.pallas import tpu as pltpu

POS_THRESH = 0.6
COORD_SCALE = 1.0


def _body(truth_ref, biases_ref, xy_ref, wh_ref, obj_ref,
          txy_ref, twh_ref, tww_ref, tobj_ref, tnoobj_ref, tlabel_ref,
          bw_scr, bh_scr):
    H, W = xy_ref.shape[2], xy_ref.shape[3]
    A = xy_ref.shape[1] // 2
    T = truth_ref.shape[2] // 5

    row_i = lax.broadcasted_iota(jnp.int32, (H, W), 0)
    col_i = lax.broadcasted_iota(jnp.int32, (H, W), 1)
    ii = col_i.astype(jnp.float32)
    jj = row_i.astype(jnp.float32)
    zero = jnp.zeros((H, W), jnp.float32)

    for a in range(A):
        x = xy_ref[0, a]
        y = xy_ref[0, a + A]
        w = wh_ref[0, a]
        h = wh_ref[0, a + A]
        bx = (x + ii) / W
        by = (y + jj) / H
        bw = jnp.exp(w) * biases_ref[a, 0] / W
        bh = jnp.exp(h) * biases_ref[a, 1] / H
        bw_scr[a] = bw
        bh_scr[a] = bh
        a1 = bw * bh
        bxl = bx - bw / 2
        bxr = bx + bw / 2
        byt = by - bh / 2
        byb = by + bh / 2

        def g_body(g, miou):
            tx = truth_ref[0, 0, 5 * g]
            ty = truth_ref[0, 0, 5 * g + 1]
            tw = truth_ref[0, 0, 5 * g + 2]
            th = truth_ref[0, 0, 5 * g + 3]
            il = jnp.maximum(bxl, tx - tw / 2)
            ir = jnp.minimum(bxr, tx + tw / 2)
            it = jnp.maximum(byt, ty - th / 2)
            ib = jnp.minimum(byb, ty + th / 2)
            ov = jnp.maximum(ir - il, 0.0) * jnp.maximum(ib - it, 0.0)
            iou = ov / (a1 + tw * th - ov)
            return jnp.maximum(miou, iou)

        miou = lax.fori_loop(0, T, g_body, zero)
        o = obj_ref[0, a]
        tnoobj_ref[0, a] = jnp.where(miou > POS_THRESH, o, 0.0)
        tobj_ref[0, a] = o
        tlabel_ref[0, a] = zero - 1.0
        txy_ref[0, a] = x
        txy_ref[0, a + A] = y
        twh_ref[0, a] = w
        twh_ref[0, a + A] = h
        tww_ref[0, a] = zero
        tww_ref[0, a + A] = zero

    def s_body(g, carry):
        tx = truth_ref[0, 0, 5 * g]
        ty = truth_ref[0, 0, 5 * g + 1]
        tw = truth_ref[0, 0, 5 * g + 2]
        th = truth_ref[0, 0, 5 * g + 3]
        cls = truth_ref[0, 0, 5 * g + 4]

        ti = (tx * W).astype(jnp.int32)
        tj = (ty * H).astype(jnp.int32)
        ti = jnp.where(ti >= W, W, ti)
        tj = jnp.where(tj >= H, H, tj)
        inval = (tx <= 0) | (tx >= 1) | (ty <= 0) | (ty >= 1)
        ti = jnp.where(inval, -1, ti)
        tj = jnp.where(inval, -1, tj)

        best = jnp.float32(-jnp.inf)
        n = jnp.int32(0)
        for a in range(A):
            b0 = biases_ref[a, 0]
            b1 = biases_ref[a, 1]
            il2 = jnp.maximum(-b0 / 2 / W, -tw / 2)
            ir2 = jnp.minimum(b0 / 2 / W, tw / 2)
            it2 = jnp.maximum(-b1 / 2 / H, -th / 2)
            ib2 = jnp.minimum(b1 / 2 / H, th / 2)
            ov2 = jnp.maximum(ir2 - il2, 0.0) * jnp.maximum(ib2 - it2, 0.0)
            iou2 = ov2 / (b0 * b1 / W / H + tw * th - ov2)
            take = iou2 > best
            n = jnp.where(take, a, n)
            best = jnp.where(take, iou2, best)
        tn = jnp.where(inval, -1, n)

        valid = (ti >= 0) & (tj >= 0) & (tj < H) & (ti < W) & (tw > 0) & (th > 0)
        ic = jnp.clip(ti, 0, W - 1)
        jc = jnp.clip(tj, 0, H - 1)
        nc = jnp.clip(tn, 0, A - 1)

        @pl.when(valid)
        def _():
            jr = pl.multiple_of((jc // 8) * 8, 8)
            js = jc - jr
            r8 = lax.broadcasted_iota(jnp.int32, (8, W), 0)
            c8 = lax.broadcasted_iota(jnp.int32, (8, W), 1)
            cmask = (r8 == js) & (c8 == ic)
            fi = ic.astype(jnp.float32)
            fj = jc.astype(jnp.float32)
            b0n = biases_ref[nc, 0]
            b1n = biases_ref[nc, 1]
            v_x = tx * W - fi
            v_y = ty * H - fj
            v_w = jnp.log(tw * W / b0n)
            v_h = jnp.log(th * H / b1n)
            wgt = COORD_SCALE * (2.0 - tw * th)

            def extract(tile):
                return jnp.sum(jnp.where(cmask, tile, 0.0))

            x_s = extract(xy_ref[0, nc, pl.ds(jr, 8), :])
            y_s = extract(xy_ref[0, nc + A, pl.ds(jr, 8), :])
            bw_s = extract(bw_scr[nc, pl.ds(jr, 8), :])
            bh_s = extract(bh_scr[nc, pl.ds(jr, 8), :])
            o_s = extract(obj_ref[0, nc, pl.ds(jr, 8), :])
            bx_s = (x_s + fi) / W
            by_s = (y_s + fj) / H
            il = jnp.maximum(bx_s - bw_s / 2, tx - tw / 2)
            ir = jnp.minimum(bx_s + bw_s / 2, tx + tw / 2)
            it = jnp.maximum(by_s - bh_s / 2, ty - th / 2)
            ib = jnp.minimum(by_s + bh_s / 2, ty + th / 2)
            ov = jnp.maximum(ir - il, 0.0) * jnp.maximum(ib - it, 0.0)
            iou_s = ov / (bw_s * bh_s + tw * th - ov)

            def put(ref, ch, val):
                ref[0, ch, pl.ds(jr, 8), :] = jnp.where(
                    cmask, val, ref[0, ch, pl.ds(jr, 8), :])

            put(txy_ref, nc, v_x)
            put(txy_ref, nc + A, v_y)
            put(twh_ref, nc, v_w)
            put(twh_ref, nc + A, v_h)
            put(tww_ref, nc, wgt)
            put(tww_ref, nc + A, wgt)
            put(tobj_ref, nc, iou_s)
            put(tnoobj_ref, nc, o_s)
            put(tlabel_ref, nc, cls)

        return carry

    lax.fori_loop(0, T, s_body, jnp.int32(0))


def _build(B, A, H, W, T, interpret=False):
    A2 = 2 * A
    big = lambda c: pl.BlockSpec((1, c, H, W), lambda b: (b, 0, 0, 0))
    in_specs = [
        pl.BlockSpec((1, 1, 5 * T), lambda b: (b, 0, 0), memory_space=pltpu.SMEM),
        pl.BlockSpec((A, 2), lambda b: (0, 0), memory_space=pltpu.SMEM),
        big(A2), big(A2), big(A),
    ]
    out_specs = [big(A2), big(A2), big(A2), big(A), big(A), big(A)]
    shp = lambda c: jax.ShapeDtypeStruct((B, c, H, W), jnp.float32)
    out_shape = [shp(A2), shp(A2), shp(A2), shp(A), shp(A), shp(A)]
    scratch = [pltpu.VMEM((A, H, W), jnp.float32)] * 2
    return pl.pallas_call(
        _body,
        grid=(B,),
        in_specs=in_specs,
        out_specs=out_specs,
        out_shape=out_shape,
        scratch_shapes=scratch,
        compiler_params=pltpu.CompilerParams(
            dimension_semantics=("arbitrary",)),
        interpret=interpret,
    )


def kernel(xy, wh, obj, truth, biases):
    xy = lax.stop_gradient(xy)
    wh = lax.stop_gradient(wh)
    obj = lax.stop_gradient(obj)
    B, A2, H, W = xy.shape
    A = A2 // 2
    T = truth.shape[1] // 5
    call = _build(B, A, H, W, T)
    return call(truth.reshape(B, 1, 5 * T), biases, xy, wh, obj)


# speedup vs baseline: 1.6169x; 1.6169x over previous
import jax
import jax.numpy as jnp
from jax import lax
from jax.experimental import pallas as pl
from jax.experimental.pallas import tpu as pltpu

POS_THRESH = 0.6
COORD_SCALE = 1.0
GL = 32


def _dot(a, b):
    return lax.dot_general(
        a, b, (((1,), (0,)), ((), ())),
        precision=lax.Precision.HIGHEST,
        preferred_element_type=jnp.float32)


def _body(truth_ref, trv_ref, biases_ref, xy_ref, wh_ref, obj_ref,
          txy_ref, twh_ref, tww_ref, tobj_ref, tnoobj_ref, tlabel_ref):
    H, W = xy_ref.shape[2], xy_ref.shape[3]
    A = xy_ref.shape[1] // 2
    T = truth_ref.shape[2] // 5

    row_i = lax.broadcasted_iota(jnp.int32, (H, W), 0)
    col_i = lax.broadcasted_iota(jnp.int32, (H, W), 1)
    ii = col_i.astype(jnp.float32)
    jj = row_i.astype(jnp.float32)
    zero = jnp.zeros((H, W), jnp.float32)

    tx = trv_ref[0, 0:1, :]
    ty = trv_ref[0, 1:2, :]
    tw = trv_ref[0, 2:3, :]
    th = trv_ref[0, 3:4, :]
    cls = trv_ref[0, 4:5, :]

    ti = (tx * W).astype(jnp.int32)
    tj = (ty * H).astype(jnp.int32)
    ti = jnp.where(ti >= W, W, ti)
    tj = jnp.where(tj >= H, H, tj)
    inval = (tx <= 0) | (tx >= 1) | (ty <= 0) | (ty >= 1)
    ti = jnp.where(inval, -1, ti)
    tj = jnp.where(inval, -1, tj)

    best = jnp.full((1, GL), -jnp.inf, jnp.float32)
    nvec = jnp.zeros((1, GL), jnp.int32)
    for a in range(A):
        b0 = biases_ref[a, 0]
        b1 = biases_ref[a, 1]
        il2 = jnp.maximum(-b0 / 2 / W, -tw / 2)
        ir2 = jnp.minimum(b0 / 2 / W, tw / 2)
        it2 = jnp.maximum(-b1 / 2 / H, -th / 2)
        ib2 = jnp.minimum(b1 / 2 / H, th / 2)
        ov2 = jnp.maximum(ir2 - il2, 0.0) * jnp.maximum(ib2 - it2, 0.0)
        iou2 = ov2 / (b0 * b1 / W / H + tw * th - ov2)
        take = iou2 > best
        nvec = jnp.where(take, a, nvec)
        best = jnp.where(take, iou2, best)
    tn = jnp.where(inval, -1, nvec)

    valid = (ti >= 0) & (tj >= 0) & (tj < H) & (ti < W) & (tw > 0) & (th > 0)
    icv = jnp.clip(ti, 0, W - 1)
    jcv = jnp.clip(tj, 0, H - 1)
    ncv = jnp.clip(tn, 0, A - 1)
    fiv = icv.astype(jnp.float32)
    fjv = jcv.astype(jnp.float32)

    keyf = ((ncv * H + jcv) * W + icv + 1).astype(jnp.float32)
    key_l = jnp.where(valid, keyf, -2.0)
    key_for_t = jnp.where(valid, keyf, -1.0)
    sub_l = lax.broadcasted_iota(jnp.int32, (H, GL), 1)
    diag = (lax.broadcasted_iota(jnp.int32, (H, GL), 0) == sub_l)
    ones_col = jnp.ones((GL, 1), jnp.float32)

    def transpose_lanes(v):
        m = jnp.where(diag, jnp.broadcast_to(v, (H, GL)), 0.0)
        return _dot(m, ones_col)

    key_s = transpose_lanes(key_for_t)
    g_sub = lax.broadcasted_iota(jnp.int32, (H, GL), 0)
    killer = (jnp.broadcast_to(key_s, (H, GL)) == jnp.broadcast_to(key_l, (H, GL))) \
        & (g_sub > sub_l)
    killed = jnp.sum(jnp.where(killer, 1.0, 0.0), axis=0, keepdims=True) > 0.5
    alive = valid & (~killed)

    rowm = jnp.where(
        lax.broadcasted_iota(jnp.int32, (H, GL), 0)
        == jnp.broadcast_to(jcv, (H, GL)), 1.0, 0.0)
    colm = jnp.where(
        lax.broadcasted_iota(jnp.int32, (W, GL), 0)
        == jnp.broadcast_to(icv, (W, GL)), 1.0, 0.0)
    ic_s = transpose_lanes(fiv)
    colmT = jnp.where(
        jnp.broadcast_to(ic_s, (H, W))
        == lax.broadcasted_iota(jnp.int32, (H, W), 1).astype(jnp.float32),
        1.0, 0.0)[0:GL, :]

    s_x = jnp.zeros((1, GL), jnp.float32)
    s_y = jnp.zeros((1, GL), jnp.float32)
    s_bw = jnp.zeros((1, GL), jnp.float32)
    s_bh = jnp.zeros((1, GL), jnp.float32)
    for a in range(A):
        x = xy_ref[0, a]
        y = xy_ref[0, a + A]
        w = wh_ref[0, a]
        h = wh_ref[0, a + A]
        bx = (x + ii) / W
        by = (y + jj) / H
        bw = jnp.exp(w) * biases_ref[a, 0] / W
        bh = jnp.exp(h) * biases_ref[a, 1] / H
        a1 = bw * bh
        bxl = bx - bw / 2
        bxr = bx + bw / 2
        byt = by - bh / 2
        byb = by + bh / 2

        def g_body(g, miou):
            gtx = truth_ref[0, 0, 5 * g]
            gty = truth_ref[0, 0, 5 * g + 1]
            gtw = truth_ref[0, 0, 5 * g + 2]
            gth = truth_ref[0, 0, 5 * g + 3]
            il = jnp.maximum(bxl, gtx - gtw / 2)
            ir = jnp.minimum(bxr, gtx + gtw / 2)
            it = jnp.maximum(byt, gty - gth / 2)
            ib = jnp.minimum(byb, gty + gth / 2)
            ov = jnp.maximum(ir - il, 0.0) * jnp.maximum(ib - it, 0.0)
            iou = ov / (a1 + gtw * gth - ov)
            return jnp.maximum(miou, iou)

        miou = lax.fori_loop(0, T, g_body, zero)
        tnoobj_ref[0, a] = jnp.where(miou > POS_THRESH, obj_ref[0, a], 0.0)

        stack = jnp.concatenate([x, y, bw, bh], axis=0)
        m1 = _dot(stack, colm)
        in_a = ncv == a
        s_x = jnp.where(in_a, jnp.sum(rowm * m1[0:H], 0, keepdims=True), s_x)
        s_y = jnp.where(in_a, jnp.sum(rowm * m1[H:2 * H], 0, keepdims=True), s_y)
        s_bw = jnp.where(in_a, jnp.sum(rowm * m1[2 * H:3 * H], 0, keepdims=True), s_bw)
        s_bh = jnp.where(in_a, jnp.sum(rowm * m1[3 * H:4 * H], 0, keepdims=True), s_bh)

    b0n = jnp.zeros((1, GL), jnp.float32)
    b1n = jnp.zeros((1, GL), jnp.float32)
    for a in range(A):
        in_a = ncv == a
        b0n = jnp.where(in_a, biases_ref[a, 0], b0n)
        b1n = jnp.where(in_a, biases_ref[a, 1], b1n)
    v_x = tx * W - fiv
    v_y = ty * H - fjv
    safe_tw = jnp.where(alive, tw, 1.0)
    safe_th = jnp.where(alive, th, 1.0)
    v_w = jnp.log(safe_tw * W / jnp.where(alive, b0n, 1.0))
    v_h = jnp.log(safe_th * H / jnp.where(alive, b1n, 1.0))
    wgt = COORD_SCALE * (2.0 - tw * th)

    bx_s = (s_x + fiv) / W
    by_s = (s_y + fjv) / H
    il = jnp.maximum(bx_s - s_bw / 2, tx - tw / 2)
    ir = jnp.minimum(bx_s + s_bw / 2, tx + tw / 2)
    it = jnp.maximum(by_s - s_bh / 2, ty - th / 2)
    ib = jnp.minimum(by_s + s_bh / 2, ty + th / 2)
    ov = jnp.maximum(ir - il, 0.0) * jnp.maximum(ib - it, 0.0)
    den = s_bw * s_bh + tw * th - ov
    iou_s = ov / jnp.where(alive, den, 1.0)

    alive_f = jnp.where(alive, 1.0, 0.0)
    vals = [jnp.where(alive, v, 0.0) * alive_f
            for v in (v_x, v_y, v_w, v_h, wgt, iou_s, cls)]
    vals.append(alive_f)

    for a in range(A):
        sel = jnp.where(ncv == a, 1.0, 0.0) * alive_f
        astack = jnp.concatenate(
            [rowm * jnp.broadcast_to(sel * v, (H, GL)) for v in vals], axis=0)
        planes = _dot(astack, colmT)
        Vx, Vy, Vw, Vh, Vwgt, Viou, Vcls, Vhit = (
            planes[k * H:(k + 1) * H] for k in range(8))
        hit = Vhit > 0.5
        txy_ref[0, a] = jnp.where(hit, Vx, xy_ref[0, a])
        txy_ref[0, a + A] = jnp.where(hit, Vy, xy_ref[0, a + A])
        twh_ref[0, a] = jnp.where(hit, Vw, wh_ref[0, a])
        twh_ref[0, a + A] = jnp.where(hit, Vh, wh_ref[0, a + A])
        tww_ref[0, a] = jnp.where(hit, Vwgt, 0.0)
        tww_ref[0, a + A] = jnp.where(hit, Vwgt, 0.0)
        o = obj_ref[0, a]
        tobj_ref[0, a] = jnp.where(hit, Viou, o)
        tnoobj_ref[0, a] = jnp.where(hit, o, tnoobj_ref[0, a])
        tlabel_ref[0, a] = jnp.where(hit, Vcls, zero - 1.0)


def _build(B, A, H, W, T, interpret=False):
    A2 = 2 * A
    big = lambda c: pl.BlockSpec((1, c, H, W), lambda b: (b, 0, 0, 0))
    in_specs = [
        pl.BlockSpec((1, 1, 5 * T), lambda b: (b, 0, 0), memory_space=pltpu.SMEM),
        pl.BlockSpec((1, 8, GL), lambda b: (b, 0, 0)),
        pl.BlockSpec((A, 2), lambda b: (0, 0), memory_space=pltpu.SMEM),
        big(A2), big(A2), big(A),
    ]
    out_specs = [big(A2), big(A2), big(A2), big(A), big(A), big(A)]
    shp = lambda c: jax.ShapeDtypeStruct((B, c, H, W), jnp.float32)
    out_shape = [shp(A2), shp(A2), shp(A2), shp(A), shp(A), shp(A)]
    return pl.pallas_call(
        _body,
        grid=(B,),
        in_specs=in_specs,
        out_specs=out_specs,
        out_shape=out_shape,
        compiler_params=pltpu.CompilerParams(
            dimension_semantics=("arbitrary",)),
        interpret=interpret,
    )


def kernel(xy, wh, obj, truth, biases):
    xy = lax.stop_gradient(xy)
    wh = lax.stop_gradient(wh)
    obj = lax.stop_gradient(obj)
    B, A2, H, W = xy.shape
    A = A2 // 2
    T = truth.shape[1] // 5
    truth5 = truth.reshape(B, T, 5).transpose(0, 2, 1)
    trv = jnp.zeros((B, 8, GL), jnp.float32).at[:, :5, :T].set(truth5)
    call = _build(B, A, H, W, T)
    return call(truth.reshape(B, 1, 5 * T), trv, biases, xy, wh, obj)


# speedup vs baseline: 2.0390x; 1.2611x over previous
import jax
import jax.numpy as jnp
from jax import lax
from jax.experimental import pallas as pl
from jax.experimental.pallas import tpu as pltpu

POS_THRESH = 0.6
COORD_SCALE = 1.0
GL = 32


def _dot(a, b):
    return lax.dot_general(
        a, b, (((1,), (0,)), ((), ())),
        precision=lax.Precision.HIGHEST,
        preferred_element_type=jnp.float32)


def _body(truth_ref, trv_ref, biases_ref, xy_ref, wh_ref, obj_ref,
          txy_ref, twh_ref, tww_ref, tobj_ref, tnoobj_ref, tlabel_ref):
    H, W = xy_ref.shape[2], xy_ref.shape[3]
    A = xy_ref.shape[1] // 2
    T = truth_ref.shape[2] // 5

    row_i = lax.broadcasted_iota(jnp.int32, (H, W), 0)
    col_i = lax.broadcasted_iota(jnp.int32, (H, W), 1)
    ii = col_i.astype(jnp.float32)
    jj = row_i.astype(jnp.float32)
    zero = jnp.zeros((H, W), jnp.float32)

    tx = trv_ref[0, 0:1, :]
    ty = trv_ref[0, 1:2, :]
    tw = trv_ref[0, 2:3, :]
    th = trv_ref[0, 3:4, :]
    cls = trv_ref[0, 4:5, :]

    ti = (tx * W).astype(jnp.int32)
    tj = (ty * H).astype(jnp.int32)
    ti = jnp.where(ti >= W, W, ti)
    tj = jnp.where(tj >= H, H, tj)
    inval = (tx <= 0) | (tx >= 1) | (ty <= 0) | (ty >= 1)
    ti = jnp.where(inval, -1, ti)
    tj = jnp.where(inval, -1, tj)

    best = jnp.full((1, GL), -jnp.inf, jnp.float32)
    nvec = jnp.zeros((1, GL), jnp.int32)
    for a in range(A):
        b0 = biases_ref[a, 0]
        b1 = biases_ref[a, 1]
        il2 = jnp.maximum(-b0 / 2 / W, -tw / 2)
        ir2 = jnp.minimum(b0 / 2 / W, tw / 2)
        it2 = jnp.maximum(-b1 / 2 / H, -th / 2)
        ib2 = jnp.minimum(b1 / 2 / H, th / 2)
        ov2 = jnp.maximum(ir2 - il2, 0.0) * jnp.maximum(ib2 - it2, 0.0)
        iou2 = ov2 / (b0 * b1 / W / H + tw * th - ov2)
        take = iou2 > best
        nvec = jnp.where(take, a, nvec)
        best = jnp.where(take, iou2, best)
    tn = jnp.where(inval, -1, nvec)

    valid = (ti >= 0) & (tj >= 0) & (tj < H) & (ti < W) & (tw > 0) & (th > 0)
    icv = jnp.clip(ti, 0, W - 1)
    jcv = jnp.clip(tj, 0, H - 1)
    ncv = jnp.clip(tn, 0, A - 1)
    fiv = icv.astype(jnp.float32)
    fjv = jcv.astype(jnp.float32)

    keyf = ((ncv * H + jcv) * W + icv + 1).astype(jnp.float32)
    key_l = jnp.where(valid, keyf, -2.0)
    key_for_t = jnp.where(valid, keyf, -1.0)
    sub_l = lax.broadcasted_iota(jnp.int32, (H, GL), 1)
    diag = (lax.broadcasted_iota(jnp.int32, (H, GL), 0) == sub_l)
    ones_col = jnp.ones((GL, 1), jnp.float32)

    def transpose_lanes(v):
        m = jnp.where(diag, jnp.broadcast_to(v, (H, GL)), 0.0)
        return _dot(m, ones_col)

    key_s = transpose_lanes(key_for_t)
    g_sub = lax.broadcasted_iota(jnp.int32, (H, GL), 0)
    killer = (jnp.broadcast_to(key_s, (H, GL)) == jnp.broadcast_to(key_l, (H, GL))) \
        & (g_sub > sub_l)
    killed = jnp.sum(jnp.where(killer, 1.0, 0.0), axis=0, keepdims=True) > 0.5
    alive = valid & (~killed)

    rowm = jnp.where(
        lax.broadcasted_iota(jnp.int32, (H, GL), 0)
        == jnp.broadcast_to(jcv, (H, GL)), 1.0, 0.0)
    colm = jnp.where(
        lax.broadcasted_iota(jnp.int32, (W, GL), 0)
        == jnp.broadcast_to(icv, (W, GL)), 1.0, 0.0)
    ic_s = transpose_lanes(fiv)
    colmT = jnp.where(
        jnp.broadcast_to(ic_s, (H, W))
        == lax.broadcasted_iota(jnp.int32, (H, W), 1).astype(jnp.float32),
        1.0, 0.0)[0:GL, :]

    s_x = jnp.zeros((1, GL), jnp.float32)
    s_y = jnp.zeros((1, GL), jnp.float32)
    s_bw = jnp.zeros((1, GL), jnp.float32)
    s_bh = jnp.zeros((1, GL), jnp.float32)
    for a in range(A):
        x = xy_ref[0, a]
        y = xy_ref[0, a + A]
        w = wh_ref[0, a]
        h = wh_ref[0, a + A]
        bx = (x + ii) / W
        by = (y + jj) / H
        bw = jnp.exp(w) * biases_ref[a, 0] / W
        bh = jnp.exp(h) * biases_ref[a, 1] / H
        a1 = bw * bh
        bxl = bx - bw / 2
        bxr = bx + bw / 2
        byt = by - bh / 2
        byb = by + bh / 2

        UNROLL = 3 if T % 3 == 0 else (2 if T % 2 == 0 else 1)

        def one_g(g, miou):
            gtx = truth_ref[0, 0, 5 * g]
            gty = truth_ref[0, 0, 5 * g + 1]
            gtw = truth_ref[0, 0, 5 * g + 2]
            gth = truth_ref[0, 0, 5 * g + 3]
            il = jnp.maximum(bxl, gtx - gtw / 2)
            ir = jnp.minimum(bxr, gtx + gtw / 2)
            it = jnp.maximum(byt, gty - gth / 2)
            ib = jnp.minimum(byb, gty + gth / 2)
            ov = jnp.maximum(ir - il, 0.0) * jnp.maximum(ib - it, 0.0)
            iou = ov / (a1 + gtw * gth - ov)
            return jnp.maximum(miou, iou)

        def g_body(k, miou):
            g = k * UNROLL
            for u in range(UNROLL):
                miou = one_g(g + u, miou)
            return miou

        miou = lax.fori_loop(0, T // UNROLL, g_body, zero)
        tnoobj_ref[0, a] = jnp.where(miou > POS_THRESH, obj_ref[0, a], 0.0)

        stack = jnp.concatenate([x, y, bw, bh], axis=0)
        m1 = _dot(stack, colm)
        in_a = ncv == a
        s_x = jnp.where(in_a, jnp.sum(rowm * m1[0:H], 0, keepdims=True), s_x)
        s_y = jnp.where(in_a, jnp.sum(rowm * m1[H:2 * H], 0, keepdims=True), s_y)
        s_bw = jnp.where(in_a, jnp.sum(rowm * m1[2 * H:3 * H], 0, keepdims=True), s_bw)
        s_bh = jnp.where(in_a, jnp.sum(rowm * m1[3 * H:4 * H], 0, keepdims=True), s_bh)

    b0n = jnp.zeros((1, GL), jnp.float32)
    b1n = jnp.zeros((1, GL), jnp.float32)
    for a in range(A):
        in_a = ncv == a
        b0n = jnp.where(in_a, biases_ref[a, 0], b0n)
        b1n = jnp.where(in_a, biases_ref[a, 1], b1n)
    v_x = tx * W - fiv
    v_y = ty * H - fjv
    safe_tw = jnp.where(alive, tw, 1.0)
    safe_th = jnp.where(alive, th, 1.0)
    v_w = jnp.log(safe_tw * W / jnp.where(alive, b0n, 1.0))
    v_h = jnp.log(safe_th * H / jnp.where(alive, b1n, 1.0))
    wgt = COORD_SCALE * (2.0 - tw * th)

    bx_s = (s_x + fiv) / W
    by_s = (s_y + fjv) / H
    il = jnp.maximum(bx_s - s_bw / 2, tx - tw / 2)
    ir = jnp.minimum(bx_s + s_bw / 2, tx + tw / 2)
    it = jnp.maximum(by_s - s_bh / 2, ty - th / 2)
    ib = jnp.minimum(by_s + s_bh / 2, ty + th / 2)
    ov = jnp.maximum(ir - il, 0.0) * jnp.maximum(ib - it, 0.0)
    den = s_bw * s_bh + tw * th - ov
    iou_s = ov / jnp.where(alive, den, 1.0)

    alive_f = jnp.where(alive, 1.0, 0.0)
    vals = [jnp.where(alive, v, 0.0) * alive_f
            for v in (v_x, v_y, v_w, v_h, wgt, iou_s, cls)]
    vals.append(alive_f)

    for a in range(A):
        sel = jnp.where(ncv == a, 1.0, 0.0) * alive_f
        astack = jnp.concatenate(
            [rowm * jnp.broadcast_to(sel * v, (H, GL)) for v in vals], axis=0)
        planes = _dot(astack, colmT)
        Vx, Vy, Vw, Vh, Vwgt, Viou, Vcls, Vhit = (
            planes[k * H:(k + 1) * H] for k in range(8))
        hit = Vhit > 0.5
        txy_ref[0, a] = jnp.where(hit, Vx, xy_ref[0, a])
        txy_ref[0, a + A] = jnp.where(hit, Vy, xy_ref[0, a + A])
        twh_ref[0, a] = jnp.where(hit, Vw, wh_ref[0, a])
        twh_ref[0, a + A] = jnp.where(hit, Vh, wh_ref[0, a + A])
        tww_ref[0, a] = jnp.where(hit, Vwgt, 0.0)
        tww_ref[0, a + A] = jnp.where(hit, Vwgt, 0.0)
        o = obj_ref[0, a]
        tobj_ref[0, a] = jnp.where(hit, Viou, o)
        tnoobj_ref[0, a] = jnp.where(hit, o, tnoobj_ref[0, a])
        tlabel_ref[0, a] = jnp.where(hit, Vcls, zero - 1.0)


def _build(B, A, H, W, T, interpret=False):
    A2 = 2 * A
    big = lambda c: pl.BlockSpec((1, c, H, W), lambda b: (b, 0, 0, 0))
    in_specs = [
        pl.BlockSpec((1, 1, 5 * T), lambda b: (b, 0, 0), memory_space=pltpu.SMEM),
        pl.BlockSpec((1, 8, GL), lambda b: (b, 0, 0)),
        pl.BlockSpec((A, 2), lambda b: (0, 0), memory_space=pltpu.SMEM),
        big(A2), big(A2), big(A),
    ]
    out_specs = [big(A2), big(A2), big(A2), big(A), big(A), big(A)]
    shp = lambda c: jax.ShapeDtypeStruct((B, c, H, W), jnp.float32)
    out_shape = [shp(A2), shp(A2), shp(A2), shp(A), shp(A), shp(A)]
    return pl.pallas_call(
        _body,
        grid=(B,),
        in_specs=in_specs,
        out_specs=out_specs,
        out_shape=out_shape,
        compiler_params=pltpu.CompilerParams(
            dimension_semantics=("arbitrary",)),
        interpret=interpret,
    )


def kernel(xy, wh, obj, truth, biases):
    xy = lax.stop_gradient(xy)
    wh = lax.stop_gradient(wh)
    obj = lax.stop_gradient(obj)
    B, A2, H, W = xy.shape
    A = A2 // 2
    T = truth.shape[1] // 5
    truth5 = truth.reshape(B, T, 5).transpose(0, 2, 1)
    trv = jnp.zeros((B, 8, GL), jnp.float32).at[:, :5, :T].set(truth5)
    call = _build(B, A, H, W, T)
    return call(truth.reshape(B, 1, 5 * T), trv, biases, xy, wh, obj)
